# Initial kernel scaffold; baseline (speedup 1.0000x reference)
import jax
import jax.numpy as jnp
from jax.experimental import pallas as pl


def kernel(x, w, idx):
    raise NotImplementedError("write your pallas kernel here")


# speedup vs baseline: 8.7533x; 8.7533x over previous
import jax
import jax.numpy as jnp
from jax.experimental import pallas as pl


def _onehot_body(idx_ref, out_ref):
    bb, n, m = out_ref.shape
    iota_n = jax.lax.broadcasted_iota(jnp.int32, (bb, n, m), 1)
    out_ref[...] = idx_ref[...] == iota_n


def kernel(x, w, idx):
    B, N = x.shape
    M = w.shape[1]
    idx32 = idx.astype(jnp.int32)
    BB = 64
    out = pl.pallas_call(
        _onehot_body,
        grid=(B // BB,),
        in_specs=[pl.BlockSpec((BB, 1, M), lambda i: (i, 0, 0))],
        out_specs=pl.BlockSpec((BB, N, M), lambda i: (i, 0, 0)),
        out_shape=jax.ShapeDtypeStruct((B, N, M), jnp.bool_),
    )(idx32)
    return out
